# baseline (speedup 1.0000x reference)
import jax
import jax.numpy as jnp
from jax import lax
from jax.experimental import pallas as pl
from jax.experimental.pallas import tpu as pltpu
from jax.experimental.pallas import tpu_sc as plsc

_L = 16
_NC = 2
_NS = 16
_NW = _NC * _NS
_R = 16
_D = 1024
_CHUNK = _R * _D


def _sc_add(x_hbm, pe_hbm, out_hbm, xb, pb, sem_x, sem_p, sem_o):
    wid = lax.axis_index("s") * _NC + lax.axis_index("c")
    n_x = x_hbm.shape[0]
    n_pe = pe_hbm.shape[0]
    per_w = n_x // _NW
    n_chunks = per_w // _CHUNK
    w_base = wid * per_w
    pe_base = lax.rem(w_base, n_pe)
    n_vregs = _CHUNK // _L

    def chunk(i, carry):
        base = w_base + i * _CHUNK
        pbase = pe_base + i * _CHUNK
        cx = pltpu.make_async_copy(x_hbm.at[pl.ds(base, _CHUNK)], xb, sem_x)
        cp = pltpu.make_async_copy(pe_hbm.at[pl.ds(pbase, _CHUNK)], pb, sem_p)
        cx.start()
        cp.start()
        cx.wait()
        cp.wait()

        def body(j, c):
            plsc.addupdate(xb.at[pl.ds(j * _L, _L)], pb[pl.ds(j * _L, _L)])
            return c

        lax.fori_loop(0, n_vregs, body, 0, unroll=8)

        co = pltpu.make_async_copy(xb, out_hbm.at[pl.ds(base, _CHUNK)], sem_o)
        co.start()
        co.wait()
        return carry

    lax.fori_loop(0, n_chunks, chunk, 0)


def kernel(x, pos_embed):
    batch, seq_len, d_model = x.shape
    xf = x.reshape(-1)
    pef = pos_embed[:seq_len].reshape(-1)
    mesh = plsc.VectorSubcoreMesh(core_axis_name="c", subcore_axis_name="s")
    out = pl.kernel(
        _sc_add,
        mesh=mesh,
        out_type=jax.ShapeDtypeStruct(xf.shape, xf.dtype),
        scratch_types=[
            pltpu.VMEM((_CHUNK,), jnp.float32),
            pltpu.VMEM((_CHUNK,), jnp.float32),
            pltpu.SemaphoreType.DMA,
            pltpu.SemaphoreType.DMA,
            pltpu.SemaphoreType.DMA,
        ],
    )(xf, pef)
    return out.reshape(x.shape)


# speedup vs baseline: 1.2482x; 1.2482x over previous
import jax
import jax.numpy as jnp
from jax import lax
from jax.experimental import pallas as pl
from jax.experimental.pallas import tpu as pltpu
from jax.experimental.pallas import tpu_sc as plsc

_L = 16
_NC = 2
_NS = 16
_NW = _NC * _NS
_R = 8
_D = 1024
_CHUNK = _R * _D
_NBUF = 4


def _sc_add(x_hbm, pe_hbm, out_hbm,
            xb0, xb1, xb2, xb3, pb0, pb1, pb2, pb3,
            sx0, sx1, sx2, sx3, sp0, sp1, sp2, sp3,
            so0, so1, so2, so3):
    wid = lax.axis_index("s") * _NC + lax.axis_index("c")
    n_x = x_hbm.shape[0]
    n_pe = pe_hbm.shape[0]
    per_w = n_x // _NW
    n_chunks = per_w // _CHUNK
    n_steps = n_chunks // _NBUF
    w_base = wid * per_w
    pe_base = lax.rem(w_base, n_pe)
    n_vregs = _CHUNK // _L

    xb = (xb0, xb1, xb2, xb3)
    pb = (pb0, pb1, pb2, pb3)
    sx = (sx0, sx1, sx2, sx3)
    sp = (sp0, sp1, sp2, sp3)
    so = (so0, so1, so2, so3)

    def in_copies(c, p):
        base = w_base + c * _CHUNK
        pbase = pe_base + c * _CHUNK
        cx = pltpu.make_async_copy(x_hbm.at[pl.ds(base, _CHUNK)], xb[p], sx[p])
        cp = pltpu.make_async_copy(pe_hbm.at[pl.ds(pbase, _CHUNK)], pb[p], sp[p])
        return cx, cp

    def out_copy(c, p):
        base = w_base + c * _CHUNK
        return pltpu.make_async_copy(xb[p], out_hbm.at[pl.ds(base, _CHUNK)], so[p])

    for p in range(_NBUF - 1):
        cx, cp = in_copies(p, p)
        cx.start()
        cp.start()

    def step(i, carry):
        for p in range(_NBUF):
            c = i * _NBUF + p

            pre = (p - 1) % _NBUF

            @pl.when(c + _NBUF - 1 < n_chunks)
            def _():
                @pl.when(c >= 1)
                def _():
                    out_copy(c - 1, pre).wait()
                ncx, ncp = in_copies(c + _NBUF - 1, pre)
                ncx.start()
                ncp.start()

            cxw, cpw = in_copies(c, p)
            cxw.wait()
            cpw.wait()

            def body(j, _):
                plsc.addupdate(xb[p].at[pl.ds(j * _L, _L)], pb[p][pl.ds(j * _L, _L)])
                return _

            lax.fori_loop(0, n_vregs, body, 0, unroll=8)
            out_copy(c, p).start()
        return carry

    lax.fori_loop(0, n_steps, step, 0)
    for p in range(_NBUF):
        out_copy(n_chunks - _NBUF + p, p).wait()


def kernel(x, pos_embed):
    batch, seq_len, d_model = x.shape
    xf = x.reshape(-1)
    pef = pos_embed[:seq_len].reshape(-1)
    mesh = plsc.VectorSubcoreMesh(core_axis_name="c", subcore_axis_name="s")
    out = pl.kernel(
        _sc_add,
        mesh=mesh,
        out_type=jax.ShapeDtypeStruct(xf.shape, xf.dtype),
        scratch_types=(
            [pltpu.VMEM((_CHUNK,), jnp.float32) for _ in range(2 * _NBUF)]
            + [pltpu.SemaphoreType.DMA for _ in range(3 * _NBUF)]
        ),
    )(xf, pef)
    return out.reshape(x.shape)


# speedup vs baseline: 4.8175x; 3.8594x over previous
import jax
import jax.numpy as jnp
from jax.experimental import pallas as pl

_BS = 512


def _pe_add_kernel(x_ref, pe_ref, o_ref):
    o_ref[...] = x_ref[...] + pe_ref[...][None, :, :]


def kernel(x, pos_embed):
    batch, seq_len, d_model = x.shape
    pe = pos_embed[:seq_len]
    grid = (seq_len // _BS, batch)
    return pl.pallas_call(
        _pe_add_kernel,
        grid=grid,
        in_specs=[
            pl.BlockSpec((1, _BS, d_model), lambda i, b: (b, i, 0)),
            pl.BlockSpec((_BS, d_model), lambda i, b: (i, 0)),
        ],
        out_specs=pl.BlockSpec((1, _BS, d_model), lambda i, b: (b, i, 0)),
        out_shape=jax.ShapeDtypeStruct(x.shape, x.dtype),
    )(x, pe)


# speedup vs baseline: 5.2237x; 1.0843x over previous
import jax
import jax.numpy as jnp
from jax.experimental import pallas as pl

_BS = 128


def _pe_add_kernel(x_ref, pe_ref, o_ref):
    o_ref[...] = x_ref[...] + pe_ref[...][None, :, :]


def kernel(x, pos_embed):
    batch, seq_len, d_model = x.shape
    pe = pos_embed[:seq_len]
    grid = (seq_len // _BS,)
    return pl.pallas_call(
        _pe_add_kernel,
        grid=grid,
        in_specs=[
            pl.BlockSpec((batch, _BS, d_model), lambda i: (0, i, 0)),
            pl.BlockSpec((_BS, d_model), lambda i: (i, 0)),
        ],
        out_specs=pl.BlockSpec((batch, _BS, d_model), lambda i: (0, i, 0)),
        out_shape=jax.ShapeDtypeStruct(x.shape, x.dtype),
    )(x, pe)


# speedup vs baseline: 5.5214x; 1.0570x over previous
import jax
import jax.numpy as jnp
from jax import lax
from jax.experimental import pallas as pl

_RB = 1024


def _pe_add_kernel(x_ref, pe_ref, o_ref):
    i = pl.program_id(0)
    pe_rows = pe_ref.shape[0]
    start = lax.rem(i * _RB, pe_rows)
    o_ref[...] = x_ref[...] + pe_ref[pl.ds(start, _RB), :]


def kernel(x, pos_embed):
    batch, seq_len, d_model = x.shape
    pe = pos_embed[:seq_len]
    xf = x.reshape(batch * seq_len, d_model)
    grid = ((batch * seq_len) // _RB,)
    out = pl.pallas_call(
        _pe_add_kernel,
        grid=grid,
        in_specs=[
            pl.BlockSpec((_RB, d_model), lambda i: (i, 0)),
            pl.BlockSpec((seq_len, d_model), lambda i: (0, 0)),
        ],
        out_specs=pl.BlockSpec((_RB, d_model), lambda i: (i, 0)),
        out_shape=jax.ShapeDtypeStruct(xf.shape, xf.dtype),
    )(xf, pe)
    return out.reshape(x.shape)


# speedup vs baseline: 5.5700x; 1.0088x over previous
import jax
import jax.numpy as jnp
from jax.experimental import pallas as pl

_BS = 512


def _pe_add_kernel(x_ref, pe_ref, o_ref):
    o_ref[...] = x_ref[...] + pe_ref[...][None, :, :]


def kernel(x, pos_embed):
    batch, seq_len, d_model = x.shape
    pe = pos_embed[:seq_len]
    grid = (seq_len // _BS,)
    return pl.pallas_call(
        _pe_add_kernel,
        grid=grid,
        in_specs=[
            pl.BlockSpec((batch, _BS, d_model), lambda i: (0, i, 0)),
            pl.BlockSpec((_BS, d_model), lambda i: (i, 0)),
        ],
        out_specs=pl.BlockSpec((batch, _BS, d_model), lambda i: (0, i, 0)),
        out_shape=jax.ShapeDtypeStruct(x.shape, x.dtype),
    )(x, pe)
